# Initial kernel scaffold; baseline (speedup 1.0000x reference)
import jax
import jax.numpy as jnp
from jax.experimental import pallas as pl


def kernel(x, grid):
    raise NotImplementedError("write your pallas kernel here")


# speedup vs baseline: 6.3489x; 6.3489x over previous
import functools

import jax
import jax.numpy as jnp
from jax import lax
from jax.experimental import pallas as pl
from jax.experimental.pallas import tpu as pltpu
from jax.experimental.pallas import tpu_sc as plsc

V = 128
D = 32
P = 500000
B = 512000
NC, NS = 2, 16
NW = NC * NS
PPW = B // NW
C = 128
NCHUNK = PPW // C


def _body(xt_hbm, grid_hbm, out_hbm, xv, idx_v, w_v, rows_v, out_v, sem):
    cid = lax.axis_index("c")
    sid = lax.axis_index("s")
    wid = sid * NC + cid
    base0 = wid * PPW

    def chunk(i, carry):
        base = base0 + i * C
        pltpu.sync_copy(xt_hbm.at[:, pl.ds(base, C)], xv)
        for g in range(C // 16):
            sl = pl.ds(g * 16, 16)
            tx = (xv[0, sl] + 2.0) * 32.0
            ty = (xv[1, sl] + 2.0) * 32.0
            tz = (xv[2, sl] + 2.0) * 32.0
            ix = tx.astype(jnp.int32)
            iy = ty.astype(jnp.int32)
            iz = tz.astype(jnp.int32)
            wx1 = tx - ix.astype(jnp.float32)
            wy1 = ty - iy.astype(jnp.float32)
            wz1 = tz - iz.astype(jnp.float32)
            wxs = (1.0 - wx1, wx1)
            wys = (1.0 - wy1, wy1)
            wzs = (1.0 - wz1, wz1)
            flat = ix + iy * V + iz * (V * V)
            for k in range(8):
                kx, ky, kz = k & 1, (k >> 1) & 1, k >> 2
                idx_v[k, sl] = flat + (kx + ky * V + kz * V * V)
                w_v[k, sl] = wxs[kx] * wys[ky] * wzs[kz]
        cps = [pltpu.async_copy(grid_hbm.at[idx_v.at[k]], rows_v.at[k], sem)
               for k in range(8)]
        for cp in cps:
            cp.wait()

        def interp(g, c2):
            p0 = g * 16
            wv = [w_v[k, pl.ds(p0, 16)] for k in range(8)]
            for j in range(16):
                acc0 = jnp.zeros((16,), jnp.float32)
                acc1 = jnp.zeros((16,), jnp.float32)
                for k in range(8):
                    wb = jnp.full((16,), wv[k][j], jnp.float32)
                    acc0 = acc0 + wb * rows_v[k, p0 + j, pl.ds(0, 16)]
                    acc1 = acc1 + wb * rows_v[k, p0 + j, pl.ds(16, 16)]
                out_v[p0 + j, pl.ds(0, 16)] = acc0
                out_v[p0 + j, pl.ds(16, 16)] = acc1
            return c2

        lax.fori_loop(0, C // 16, interp, 0)
        pltpu.sync_copy(out_v, out_hbm.at[pl.ds(base, C)])
        return carry

    lax.fori_loop(0, NCHUNK, chunk, 0)


_mesh = plsc.VectorSubcoreMesh(core_axis_name="c", subcore_axis_name="s")

_sc_call = pl.kernel(
    _body,
    out_type=jax.ShapeDtypeStruct((B, D), jnp.float32),
    mesh=_mesh,
    scratch_types=[
        pltpu.VMEM((3, C), jnp.float32),
        pltpu.VMEM((8, C), jnp.int32),
        pltpu.VMEM((8, C), jnp.float32),
        pltpu.VMEM((8, C, D), jnp.float32),
        pltpu.VMEM((C, D), jnp.float32),
        pltpu.SemaphoreType.DMA,
    ],
    compiler_params=pltpu.CompilerParams(use_tc_tiling_on_sc=False),
)


@jax.jit
def kernel(x, grid):
    pad = jnp.full((B - P, 3), 0.5, jnp.float32)
    xt = jnp.concatenate([x, pad], axis=0).T
    out = _sc_call(xt, grid)
    return out[:P]


# speedup vs baseline: 7.0645x; 1.1127x over previous
import jax
import jax.numpy as jnp
from jax import lax
from jax.experimental import pallas as pl
from jax.experimental.pallas import tpu as pltpu
from jax.experimental.pallas import tpu_sc as plsc

V = 128
D = 32
P = 500000
NC, NS = 2, 16
NW = NC * NS
C = 128
NCHUNK = 126
PPW = C * NCHUNK
B = PPW * NW


def _body(xt_hbm, grid_hbm, out_hbm, xv, idx_v, w_v, rows_v, out_v,
          sem_x, sem_g, sem_o):
    cid = lax.axis_index("c")
    sid = lax.axis_index("s")
    wid = sid * NC + cid
    base0 = wid * PPW

    def load_x(i, par):
        return pltpu.async_copy(
            xt_hbm.at[:, pl.ds(base0 + i * C, C)], xv.at[par], sem_x.at[par])

    def compute_idx_w(par):
        for g in range(C // 16):
            sl = pl.ds(g * 16, 16)
            tx = (xv[par, 0, sl] + 2.0) * 32.0
            ty = (xv[par, 1, sl] + 2.0) * 32.0
            tz = (xv[par, 2, sl] + 2.0) * 32.0
            ix = tx.astype(jnp.int32)
            iy = ty.astype(jnp.int32)
            iz = tz.astype(jnp.int32)
            wx1 = tx - ix.astype(jnp.float32)
            wy1 = ty - iy.astype(jnp.float32)
            wz1 = tz - iz.astype(jnp.float32)
            wxs = (1.0 - wx1, wx1)
            wys = (1.0 - wy1, wy1)
            wzs = (1.0 - wz1, wz1)
            flat = ix + iy * V + iz * (V * V)
            for k in range(8):
                kx, ky, kz = k & 1, (k >> 1) & 1, k >> 2
                idx_v[par, k, sl] = flat + (kx + ky * V + kz * V * V)
                w_v[par, k, sl] = wxs[kx] * wys[ky] * wzs[kz]

    def fire_gathers(par):
        for k in range(8):
            pltpu.async_copy(grid_hbm.at[idx_v.at[par, k]],
                             rows_v.at[par, k], sem_g.at[par])

    def wait_gathers(par):
        for k in range(8):
            pltpu.make_async_copy(grid_hbm.at[idx_v.at[par, k]],
                                  rows_v.at[par, k], sem_g.at[par]).wait()

    def interp(par):
        def group(g, c2):
            p0 = g * 16
            wv = [w_v[par, k, pl.ds(p0, 16)] for k in range(8)]
            for j in range(16):
                acc0 = jnp.zeros((16,), jnp.float32)
                acc1 = jnp.zeros((16,), jnp.float32)
                for k in range(8):
                    wb = jnp.full((16,), wv[k][j], jnp.float32)
                    acc0 = acc0 + wb * rows_v[par, k, p0 + j, pl.ds(0, 16)]
                    acc1 = acc1 + wb * rows_v[par, k, p0 + j, pl.ds(16, 16)]
                out_v[par, p0 + j, pl.ds(0, 16)] = acc0
                out_v[par, p0 + j, pl.ds(16, 16)] = acc1
            return c2

        lax.fori_loop(0, C // 16, group, 0)

    def store_out(i, par):
        return pltpu.async_copy(
            out_v.at[par], out_hbm.at[pl.ds(base0 + i * C, C)], sem_o.at[par])

    load_x(0, 0).wait()
    compute_idx_w(0)
    fire_gathers(0)
    load_x(1, 1)

    def chunk(i, carry):
        par = lax.rem(i, 2)
        nxt = 1 - par

        @pl.when(i + 1 < NCHUNK)
        def _():
            pltpu.make_async_copy(
                xt_hbm.at[:, pl.ds(base0 + (i + 1) * C, C)], xv.at[nxt],
                sem_x.at[nxt]).wait()
            compute_idx_w(nxt)
            fire_gathers(nxt)

        @pl.when(i + 2 < NCHUNK)
        def _():
            load_x(i + 2, par)

        @pl.when(i >= 2)
        def _():
            pltpu.make_async_copy(
                out_v.at[par], out_hbm.at[pl.ds(base0 + (i - 2) * C, C)],
                sem_o.at[par]).wait()

        wait_gathers(par)
        interp(par)
        store_out(i, par)
        return carry

    lax.fori_loop(0, NCHUNK, chunk, 0)

    for i in (NCHUNK - 2, NCHUNK - 1):
        par = i % 2
        pltpu.make_async_copy(
            out_v.at[par], out_hbm.at[pl.ds(base0 + i * C, C)],
            sem_o.at[par]).wait()


_mesh = plsc.VectorSubcoreMesh(core_axis_name="c", subcore_axis_name="s")

_sc_call = pl.kernel(
    _body,
    out_type=jax.ShapeDtypeStruct((B, D), jnp.float32),
    mesh=_mesh,
    scratch_types=[
        pltpu.VMEM((2, 3, C), jnp.float32),
        pltpu.VMEM((2, 8, C), jnp.int32),
        pltpu.VMEM((2, 8, C), jnp.float32),
        pltpu.VMEM((2, 8, C, D), jnp.float32),
        pltpu.VMEM((2, C, D), jnp.float32),
        pltpu.SemaphoreType.DMA((2,)),
        pltpu.SemaphoreType.DMA((2,)),
        pltpu.SemaphoreType.DMA((2,)),
    ],
    compiler_params=pltpu.CompilerParams(use_tc_tiling_on_sc=False),
)


@jax.jit
def kernel(x, grid):
    pad = jnp.full((B - P, 3), 0.5, jnp.float32)
    xt = jnp.concatenate([x, pad], axis=0).T
    out = _sc_call(xt, grid)
    return out[:P]
